# baseline (speedup 1.0000x reference)
import functools

import jax
import jax.numpy as jnp
from jax import lax
from jax.experimental import pallas as pl
from jax.experimental.pallas import tpu as pltpu
from jax.experimental.pallas import tpu_sc as plsc

_B = 16384
_F = 64
_NC = 2
_NS = 16
_NW = _NC * _NS
_BPW = _B // _NW
_CHUNK = 128
_NCHUNK = _BPW // _CHUNK
_L = 16


def _sc_body(uid_hbm, iid_hbm, ut_hbm, it_hbm, out_hbm,
             uidx_v, iidx_v, urows_v, irows_v, out_v, sem):
    wid = lax.axis_index("s") * _NC + lax.axis_index("c")
    base_c = wid * _NCHUNK

    pltpu.sync_copy(uid_hbm.at[pl.ds(base_c, _NCHUNK)], uidx_v)
    pltpu.sync_copy(iid_hbm.at[pl.ds(base_c, _NCHUNK)], iidx_v)

    copies = []
    for j in range(_NCHUNK):
        copies.append(pltpu.async_copy(
            ut_hbm.at[uidx_v.at[j]], urows_v.at[pl.ds(j * _CHUNK, _CHUNK)], sem))
        copies.append(pltpu.async_copy(
            it_hbm.at[iidx_v.at[j]], irows_v.at[pl.ds(j * _CHUNK, _CHUNK)], sem))
    for c in copies:
        c.wait()

    lanes = lax.iota(jnp.int32, _L)

    @plsc.parallel_loop(0, _BPW, _L)
    def _(r0):
        res = jnp.zeros((_L,), jnp.float32)
        for i in range(_L):
            r = r0 + i
            acc = urows_v[r, pl.ds(0, _L)] * irows_v[r, pl.ds(0, _L)]
            for k in range(1, _F // _L):
                acc = acc + urows_v[r, pl.ds(k * _L, _L)] * irows_v[r, pl.ds(k * _L, _L)]
            res = jnp.where(lanes == i, jnp.sum(acc), res)
        out_v[pl.ds(r0, _L)] = res

    pltpu.sync_copy(out_v, out_hbm.at[pl.ds(wid * _BPW, _BPW)])


@jax.jit
def _cfmodel_sc(uid, iid, user_table, item_table):
    mesh = plsc.VectorSubcoreMesh(core_axis_name="c", subcore_axis_name="s")
    run = functools.partial(
        pl.kernel,
        out_type=jax.ShapeDtypeStruct((_B,), jnp.float32),
        mesh=mesh,
        scratch_types=[
            pltpu.VMEM((_NCHUNK, _CHUNK), jnp.int32),
            pltpu.VMEM((_NCHUNK, _CHUNK), jnp.int32),
            pltpu.VMEM((_BPW, _F), jnp.float32),
            pltpu.VMEM((_BPW, _F), jnp.float32),
            pltpu.VMEM((_BPW,), jnp.float32),
            pltpu.SemaphoreType.DMA,
        ],
        compiler_params=pltpu.CompilerParams(
            needs_layout_passes=False, use_tc_tiling_on_sc=False),
    )(_sc_body)
    return run(uid, iid, user_table, item_table)


def kernel(input_user_id, input_item_id, user_table, item_table):
    uid = input_user_id.reshape(_NW * _NCHUNK, _CHUNK).astype(jnp.int32)
    iid = input_item_id.reshape(_NW * _NCHUNK, _CHUNK).astype(jnp.int32)
    out = _cfmodel_sc(uid, iid, user_table, item_table)
    return out.reshape(_B, 1)


# speedup vs baseline: 1.4955x; 1.4955x over previous
import functools

import jax
import jax.numpy as jnp
from jax import lax
from jax.experimental import pallas as pl
from jax.experimental.pallas import tpu as pltpu
from jax.experimental.pallas import tpu_sc as plsc

_B = 16384
_F = 64
_NC = 2
_NS = 16
_NW = _NC * _NS
_BPW = _B // _NW
_G = 16
_NG = _BPW // _G
_RB = 8
_IDXROW = 128


def _group_ids(idx_v, g):
    j = g // (_IDXROW // _G)
    col = (g % (_IDXROW // _G)) * _G
    return idx_v[j, pl.ds(col, _G)]


def _fire_group(tab_hbm, ids_vec, bbuf, buf, sem, issue):
    copies = []
    for i in range(_G):
        r0 = pl.multiple_of((ids_vec[i] >> 3) << 3, 8)
        dst = bbuf.at[pl.ds((buf * _G + i) * _RB, _RB), :]
        src = tab_hbm.at[pl.ds(r0, _RB), :]
        if issue:
            copies.append(pltpu.async_copy(src, dst, sem))
        else:
            copies.append(pltpu.make_async_copy(src, dst, sem))
    return copies


def _sc_body(uid_hbm, iid_hbm, ut_hbm, it_hbm, out_hbm,
             uidx_v, iidx_v, ublk_v, iblk_v, out_v, usem, isem):
    wid = lax.axis_index("s") * _NC + lax.axis_index("c")
    base_c = wid * (_BPW // _IDXROW)

    pltpu.sync_copy(uid_hbm.at[pl.ds(base_c, _BPW // _IDXROW)], uidx_v)
    pltpu.sync_copy(iid_hbm.at[pl.ds(base_c, _BPW // _IDXROW)], iidx_v)

    lanes = lax.iota(jnp.int32, _G)

    _fire_group(ut_hbm, _group_ids(uidx_v, 0), ublk_v, 0, usem, True)
    _fire_group(it_hbm, _group_ids(iidx_v, 0), iblk_v, 0, isem, True)

    def group_body(g, carry):
        buf = lax.rem(g, 2)
        uids = _group_ids(uidx_v, g)
        iids = _group_ids(iidx_v, g)

        @pl.when(g + 1 < _NG)
        def _():
            _fire_group(ut_hbm, _group_ids(uidx_v, g + 1), ublk_v,
                        1 - buf, usem, True)
            _fire_group(it_hbm, _group_ids(iidx_v, g + 1), iblk_v,
                        1 - buf, isem, True)

        for c in _fire_group(ut_hbm, uids, ublk_v, buf, usem, False):
            c.wait()
        for c in _fire_group(it_hbm, iids, iblk_v, buf, isem, False):
            c.wait()

        res = jnp.zeros((_G,), jnp.float32)
        for i in range(_G):
            ur = (buf * _G + i) * _RB + (uids[i] & (_RB - 1))
            ir = (buf * _G + i) * _RB + (iids[i] & (_RB - 1))
            acc = ublk_v[ur, pl.ds(0, _G)] * iblk_v[ir, pl.ds(0, _G)]
            for k in range(1, _F // _G):
                acc = acc + (ublk_v[ur, pl.ds(k * _G, _G)]
                             * iblk_v[ir, pl.ds(k * _G, _G)])
            res = jnp.where(lanes == i, jnp.sum(acc), res)
        out_v[pl.ds(g * _G, _G)] = res
        return carry

    lax.fori_loop(0, _NG, group_body, 0, unroll=False)

    pltpu.sync_copy(out_v, out_hbm.at[pl.ds(wid * _BPW, _BPW)])


@jax.jit
def _cfmodel_sc(uid, iid, ut, it):
    mesh = plsc.VectorSubcoreMesh(core_axis_name="c", subcore_axis_name="s")
    run = functools.partial(
        pl.kernel,
        out_type=jax.ShapeDtypeStruct((_B,), jnp.float32),
        mesh=mesh,
        scratch_types=[
            pltpu.VMEM((_BPW // _IDXROW, _IDXROW), jnp.int32),
            pltpu.VMEM((_BPW // _IDXROW, _IDXROW), jnp.int32),
            pltpu.VMEM((2 * _G * _RB, _F), jnp.float32),
            pltpu.VMEM((2 * _G * _RB, _F), jnp.float32),
            pltpu.VMEM((_BPW,), jnp.float32),
            pltpu.SemaphoreType.DMA,
            pltpu.SemaphoreType.DMA,
        ],
        compiler_params=pltpu.CompilerParams(
            needs_layout_passes=False, use_tc_tiling_on_sc=True),
    )(_sc_body)
    return run(uid, iid, ut, it)


def kernel(input_user_id, input_item_id, user_table, item_table):
    uid = input_user_id.reshape(_B // _IDXROW, _IDXROW).astype(jnp.int32)
    iid = input_item_id.reshape(_B // _IDXROW, _IDXROW).astype(jnp.int32)
    out = _cfmodel_sc(uid, iid, user_table, item_table)
    return out.reshape(_B, 1)


# speedup vs baseline: 2.9877x; 1.9979x over previous
import functools

import jax
import jax.numpy as jnp
from jax import lax
from jax.experimental import pallas as pl
from jax.experimental.pallas import tpu as pltpu
from jax.experimental.pallas import tpu_sc as plsc

_B = 16384
_F = 64
_NC = 2
_NS = 16
_NW = _NC * _NS
_BPW = _B // _NW
_G = 16
_IDXROW = 128
_BATCH = 64
_STAGE_ROWS = _B + _NW * _BATCH
_RING = 4

_UBLOCKS = 7813
_UB_PER_W = 244
_UB_MAX = 249
_UCAP = 32
_IBLOCKS = 782
_IB_MAX = 25
_ICAP = 64
_LCAP = 1024


def _splat(x):
    return jnp.full((_G,), x, jnp.int32)


def _extract_body(nblk_max, cap, blo_fn, bhi_fn):
    def body(ids_hbm, tab_hbm, stage_hbm,
             idsv, listb, listid, memb, bb, rowbuf, bidx_v,
             cnt_s, nblist_s, nnb_s, bsem):
        wid = lax.axis_index("s") * _NC + lax.axis_index("c")
        blo = blo_fn(wid)
        bhi = bhi_fn(wid)
        lanes = lax.iota(jnp.int32, _G)

        pltpu.sync_copy(ids_hbm, idsv)

        def filt(t, off):
            vec = idsv[t // 8, pl.ds((t % 8) * _G, _G)]
            blk = vec >> 7
            m = (blk >= blo) & (blk < bhi)
            bvec = t * _G + lanes
            plsc.store_compressed(listb.at[pl.ds(off, _G)], bvec, mask=m)
            plsc.store_compressed(listid.at[pl.ds(off, _G)], vec, mask=m)
            return off + plsc.all_reduce_population_count(m)[0]

        count = lax.fori_loop(0, (_B // _G), filt, jnp.int32(0),
                              unroll=False)

        def zero(k, c):
            cnt_s[k] = jnp.int32(0)
            return c
        lax.fori_loop(0, nblk_max, zero, 0, unroll=False)
        nnb_s[0] = jnp.int32(0)

        def bucket(t, c):
            id16 = listid[pl.ds(t * _G, _G)]
            for i in range(_G):
                @pl.when(t * _G + i < count)
                def _():
                    k = (id16[i] >> 7) - blo
                    n = cnt_s[k]
                    cnt_s[k] = n + 1

                    @pl.when(n == 0)
                    def _():
                        j = nnb_s[0]
                        nblist_s[j] = k
                        nnb_s[0] = j + 1

                    plsc.store_scatter(
                        memb, [_splat(k * cap + n)], _splat(t * _G + i),
                        mask=lanes == 0)
            return c
        lax.fori_loop(0, (count + _G - 1) // _G, bucket, 0, unroll=False)
        nnb = nnb_s[0]

        def fetch(j):
            k = nblist_s[j]
            c0 = pl.multiple_of((blo + k) * 128, 128)
            s = lax.rem(j, _RING)
            return pltpu.make_async_copy(
                tab_hbm.at[:, pl.ds(c0, 128)],
                bb.at[pl.ds(s * _F, _F), :], bsem.at[s])

        for j0 in range(_RING):
            @pl.when(j0 < nnb)
            def _():
                fetch(j0).start()

        def park_bidx():
            for q in range(_BATCH // _G):
                bidx_v[pl.ds(q * _G, _G)] = (_B + wid * _BATCH + q * _G
                                             + lanes)

        park_bidx()

        def sweep(j, carry):
            slot = carry
            k = nblist_s[j]
            n = cnt_s[k]
            fetch(j).wait()
            rbase = lax.rem(j, _RING) * _F

            def member(m, slot2):
                ptr = plsc.load_gather(memb, [_splat(k * cap + m)])[0]
                b = plsc.load_gather(listb, [_splat(ptr)])[0]
                mid = plsc.load_gather(listid, [_splat(ptr)])[0]
                c = mid & 127
                for k4 in range(_F // _G):
                    q = plsc.load_gather(bb, [rbase + k4 * _G + lanes,
                                              _splat(c)])
                    rowbuf[slot2, pl.ds(k4 * _G, _G)] = q
                plsc.store_scatter(bidx_v, [_splat(slot2)], _splat(b),
                                   mask=lanes == 0)
                full = slot2 == (_BATCH - 1)

                @pl.when(full)
                def _():
                    pltpu.sync_copy(rowbuf, stage_hbm.at[bidx_v])
                    park_bidx()

                return jnp.where(full, 0, slot2 + 1)

            slot_out = lax.fori_loop(0, n, member, slot, unroll=False)

            @pl.when(j + _RING < nnb)
            def _():
                fetch(j + _RING).start()

            return slot_out

        lax.fori_loop(0, nnb, sweep, jnp.int32(0), unroll=False)

        pltpu.sync_copy(rowbuf, stage_hbm.at[bidx_v])

    return body


def _dot_body(ustage_hbm, istage_hbm, out_hbm, ublk, iblk, out_v, sem):
    wid = lax.axis_index("s") * _NC + lax.axis_index("c")
    lanes = lax.iota(jnp.int32, _G)
    half_rows = _BPW // 2

    for h in range(2):
        base = wid * _BPW + h * half_rows
        pltpu.sync_copy(ustage_hbm.at[pl.ds(base, half_rows)], ublk)
        pltpu.sync_copy(istage_hbm.at[pl.ds(base, half_rows)], iblk)

        def group(g, c):
            res = jnp.zeros((_G,), jnp.float32)
            for i in range(_G):
                r = g * _G + i
                acc = ublk[r, pl.ds(0, _G)] * iblk[r, pl.ds(0, _G)]
                for k in range(1, _F // _G):
                    acc = acc + (ublk[r, pl.ds(k * _G, _G)]
                                 * iblk[r, pl.ds(k * _G, _G)])
                res = jnp.where(lanes == i, jnp.sum(acc), res)
            out_v[pl.ds(h * half_rows + g * _G, _G)] = res
            return c

        lax.fori_loop(0, half_rows // _G, group, 0, unroll=False)

    pltpu.sync_copy(out_v, out_hbm.at[pl.ds(wid * _BPW, _BPW)])


def _mesh():
    return plsc.VectorSubcoreMesh(core_axis_name="c", subcore_axis_name="s")


def _extract_kernel(nblk_max, cap, blo_fn, bhi_fn):
    return functools.partial(
        pl.kernel,
        out_type=jax.ShapeDtypeStruct((_STAGE_ROWS, 128), jnp.float32),
        mesh=_mesh(),
        scratch_types=[
            pltpu.VMEM((_B // _IDXROW, _IDXROW), jnp.int32),
            pltpu.VMEM((_LCAP,), jnp.int32),
            pltpu.VMEM((_LCAP,), jnp.int32),
            pltpu.VMEM((nblk_max * cap,), jnp.int32),
            pltpu.VMEM((_RING * _F, 128), jnp.float32),
            pltpu.VMEM((_BATCH, 128), jnp.float32),
            pltpu.VMEM((_BATCH,), jnp.int32),
            pltpu.SMEM((nblk_max,), jnp.int32),
            pltpu.SMEM((nblk_max,), jnp.int32),
            pltpu.SMEM((1,), jnp.int32),
            pltpu.SemaphoreType.DMA((_RING,)),
        ],
        compiler_params=pltpu.CompilerParams(
            needs_layout_passes=False, use_tc_tiling_on_sc=True),
    )(_extract_body(nblk_max, cap, blo_fn, bhi_fn))


@jax.jit
def _cfmodel_g(uid, iid, ut_t, it_t):
    u_extract = _extract_kernel(
        _UB_MAX, _UCAP,
        lambda w: w * _UB_PER_W,
        lambda w: jnp.where(w == _NW - 1, _UBLOCKS, (w + 1) * _UB_PER_W))
    i_extract = _extract_kernel(
        _IB_MAX, _ICAP,
        lambda w: 24 * w + jnp.minimum(w, 14),
        lambda w: 24 * (w + 1) + jnp.minimum(w + 1, 14))
    ustage = u_extract(uid, ut_t)
    istage = i_extract(iid, it_t)

    dot = functools.partial(
        pl.kernel,
        out_type=jax.ShapeDtypeStruct((_B,), jnp.float32),
        mesh=_mesh(),
        scratch_types=[
            pltpu.VMEM((_BPW // 2, 128), jnp.float32),
            pltpu.VMEM((_BPW // 2, 128), jnp.float32),
            pltpu.VMEM((_BPW,), jnp.float32),
            pltpu.SemaphoreType.DMA,
        ],
        compiler_params=pltpu.CompilerParams(
            needs_layout_passes=False, use_tc_tiling_on_sc=True),
    )(_dot_body)
    return dot(ustage, istage)


def kernel(input_user_id, input_item_id, user_table, item_table):
    uid = input_user_id.reshape(_B // _IDXROW, _IDXROW).astype(jnp.int32)
    iid = input_item_id.reshape(_B // _IDXROW, _IDXROW).astype(jnp.int32)
    out = _cfmodel_g(uid, iid, user_table.T, item_table.T)
    return out.reshape(_B, 1)
